# baseline (speedup 1.0000x reference)
import functools

import jax
import jax.numpy as jnp
from jax import lax
from jax.experimental import pallas as pl
from jax.experimental.pallas import tpu as pltpu
from jax.experimental.pallas import tpu_sc as plsc

H = 8
D = 16
HD = H * D

NC = 2
NS = 16
NW = NC * NS
EPC = 32
zrows_unit = 16


def _qkvs_body(x_ref, w_ref, b_ref, q_ref, k_ref, v_ref, s_ref):
    y = jnp.dot(x_ref[...], w_ref[...], preferred_element_type=jnp.float32)
    y = y + b_ref[...]
    q_ref[...] = y[:, 0 * HD:1 * HD]
    k_ref[...] = y[:, 1 * HD:2 * HD]
    v_ref[...] = y[:, 2 * HD:3 * HD]
    s_ref[...] = y[:, 3 * HD:4 * HD]


def _qkvs(feat, wt, ball, n_block):
    n = feat.shape[0]
    grid = (n // n_block,)
    spec_x = pl.BlockSpec((n_block, HD), lambda i: (i, 0))
    spec_w = pl.BlockSpec((HD, 4 * HD), lambda i: (0, 0))
    spec_b = pl.BlockSpec((1, 4 * HD), lambda i: (0, 0))
    spec_o = pl.BlockSpec((n_block, HD), lambda i: (i, 0))
    out = pl.pallas_call(
        _qkvs_body,
        grid=grid,
        in_specs=[spec_x, spec_w, spec_b],
        out_specs=[spec_o] * 4,
        out_shape=[jax.ShapeDtypeStruct((n, HD), jnp.float32)] * 4,
    )(feat, wt, ball)
    return out


def _edge_sc(sd, q, k, v, npad, e):
    epc = EPC
    nch = sd.shape[1]
    rows_per_tile = npad // NS
    zrows = zrows_unit
    nzcop = rows_per_tile // zrows
    mesh = plsc.VectorSubcoreMesh(core_axis_name="c", subcore_axis_name="s")

    @functools.partial(
        pl.kernel,
        mesh=mesh,
        compiler_params=pltpu.CompilerParams(
            needs_layout_passes=False, use_tc_tiling_on_sc=False),
        out_type=[
            jax.ShapeDtypeStruct((npad, HD), jnp.float32),
            jax.ShapeDtypeStruct((npad, HD), jnp.float32),
            jax.ShapeDtypeStruct((npad, 16), jnp.float32),
            jax.ShapeDtypeStruct((npad, 16), jnp.float32),
        ],
        scratch_types=[
            pltpu.VMEM((nch, epc), jnp.int32),
            pltpu.VMEM((epc,), jnp.int32),
            pltpu.VMEM((2, epc), jnp.int32),
            pltpu.VMEM((epc, HD), jnp.float32),
            pltpu.VMEM((epc, HD), jnp.float32),
            pltpu.VMEM((epc, HD), jnp.float32),
            pltpu.VMEM((epc, 16), jnp.float32),
            pltpu.VMEM((zrows_unit, HD), jnp.float32),
            pltpu.VMEM((zrows_unit, 16), jnp.float32),
            pltpu.VMEM_SHARED((npad, HD), jnp.float32),
            pltpu.VMEM_SHARED((npad, 16), jnp.float32),
            pltpu.SemaphoreType.DMA,
            pltpu.SemaphoreType.DMA,
        ],
    )
    def edge_kernel(sd_hbm, q_hbm, k_hbm, v_hbm,
                    agg0_out, agg1_out, den0_out, den1_out,
                    sd_v, sidx, didx2, qrows, krows, vrows, exv, zbuf, zbufd,
                    agg_s, den_s, sem_g, sem_s):
        c = lax.axis_index("c")
        s = lax.axis_index("s")
        w = c * NS + s
        rbase = s * rows_per_tile
        z16 = jnp.zeros((16,), jnp.float32)
        z8 = jnp.zeros((8,), jnp.float32)

        def zero_body(i, carry):
            for j in range(HD // 16):
                zbuf[i, pl.ds(j * 16, 16)] = z16
            return carry

        lax.fori_loop(0, zrows, zero_body, 0)
        for kk in range(nzcop):
            sl = pl.ds(rbase + kk * zrows, zrows)
            pltpu.sync_copy(zbuf, agg_s.at[sl])
            pltpu.sync_copy(zbuf.at[pl.ds(0, zrows), pl.ds(0, 16)],
                            den_s.at[sl])
        plsc.subcore_barrier()

        pltpu.sync_copy(sd_hbm.at[w], sd_v)

        lane = lax.iota(jnp.int32, 16)
        m0 = lane == jnp.zeros((16,), jnp.int32)

        def ez_body(i, carry):
            exv[i, :] = z16
            return carry

        lax.fori_loop(0, epc, ez_body, 0)

        def chunk_body(g, carry):
            p = lax.rem(g, 2)
            didx = didx2.at[p]
            for tt in range(epc // 16):
                sv = sd_v[g, pl.ds(tt * 16, 16)]
                sidx[pl.ds(tt * 16, 16)] = lax.bitwise_and(sv, 0xFFFF)
                didx[pl.ds(tt * 16, 16)] = lax.shift_right_logical(sv, 16)
            cq = pltpu.async_copy(q_hbm.at[sidx], qrows, sem_g)
            ck = pltpu.async_copy(k_hbm.at[didx], krows, sem_g)

            @pl.when(g > 0)
            def _():
                dprev = didx2.at[1 - p]
                pltpu.make_async_copy(exv, den_s.at[dprev], sem_s).wait()
                pltpu.make_async_copy(vrows, agg_s.at[dprev], sem_s).wait()

            cv = pltpu.async_copy(v_hbm.at[sidx], vrows, sem_g)
            cq.wait()
            ck.wait()
            cv.wait()
            for ee in range(epc):
                for h in range(H):
                    qv = qrows[ee, pl.ds(h * D, D)]
                    kv = krows[ee, pl.ds(h * D, D)]
                    ev = jnp.exp(jnp.full((16,), jnp.sum(qv * kv),
                                          jnp.float32))
                    plsc.store_scatter(
                        exv,
                        [jnp.full((16,), ee, jnp.int32),
                         jnp.full((16,), h, jnp.int32)],
                        ev, mask=m0)
                    vrows[ee, pl.ds(h * D, D)] = (
                        vrows[ee, pl.ds(h * D, D)] * ev)
            pltpu.async_copy(exv, den_s.at[didx], sem_s, add=True)
            pltpu.async_copy(vrows, agg_s.at[didx], sem_s, add=True)
            return carry

        lax.fori_loop(0, nch, chunk_body, 0)
        dlast = didx2.at[(nch - 1) % 2]
        pltpu.make_async_copy(exv, den_s.at[dlast], sem_s).wait()
        pltpu.make_async_copy(vrows, agg_s.at[dlast], sem_s).wait()
        plsc.subcore_barrier()

        def dump_body(kk, carry):
            sl = pl.ds(rbase + kk * zrows, zrows)
            pltpu.sync_copy(agg_s.at[sl], zbuf)
            pltpu.sync_copy(den_s.at[sl], zbufd)

            @pl.when(c == 0)
            def _():
                pltpu.sync_copy(zbuf, agg0_out.at[sl])
                pltpu.sync_copy(zbufd, den0_out.at[sl])

            @pl.when(c == 1)
            def _():
                pltpu.sync_copy(zbuf, agg1_out.at[sl])
                pltpu.sync_copy(zbufd, den1_out.at[sl])

            return carry

        lax.fori_loop(0, nzcop, dump_body, 0)

    return edge_kernel(sd, q, k, v)


def _epi_body(skip_ref, a0_ref, a1_ref, d0_ref, d1_ref, erep_ref, wa_ref,
              wb_ref, gb_ref, gamma_ref, beta_ref, pa_ref, out_ref):
    skip = skip_ref[...]
    aggu = a0_ref[...] + a1_ref[...]
    den = d0_ref[...] + d1_ref[...]
    rec = jnp.where(den > 0.0, 1.0 / den, 0.0)
    recf = jnp.dot(rec, erep_ref[...], preferred_element_type=jnp.float32)
    agg = aggu * recf
    logit = (jnp.sum(skip * wa_ref[...], axis=-1, keepdims=True)
             + jnp.sum(agg * wb_ref[...], axis=-1, keepdims=True)
             + gb_ref[0, 0])
    gate = jax.nn.sigmoid(logit)
    rst = gate * skip + (1.0 - gate) * agg
    mu = jnp.mean(rst, axis=-1, keepdims=True)
    var = jnp.mean((rst - mu) * (rst - mu), axis=-1, keepdims=True)
    y = (rst - mu) * lax.rsqrt(var + 1e-5)
    y = y * gamma_ref[...] + beta_ref[...]
    out_ref[...] = jnp.where(y >= 0.0, y, pa_ref[0, 0] * y)


def _epilogue(skip, a0, a1, d0, d1, erep, wa, wb, gb, gamma, beta, pa, n_block,
              n_out):
    n = n_out
    grid = (n // n_block,)
    row = lambda i: (i, 0)
    full = lambda i: (0, 0)
    out = pl.pallas_call(
        _epi_body,
        grid=grid,
        in_specs=[
            pl.BlockSpec((n_block, HD), row),
            pl.BlockSpec((n_block, HD), row),
            pl.BlockSpec((n_block, HD), row),
            pl.BlockSpec((n_block, 16), row),
            pl.BlockSpec((n_block, 16), row),
            pl.BlockSpec((16, HD), full),
            pl.BlockSpec((1, HD), full),
            pl.BlockSpec((1, HD), full),
            pl.BlockSpec((1, 1), full),
            pl.BlockSpec((1, HD), full),
            pl.BlockSpec((1, HD), full),
            pl.BlockSpec((1, 1), full),
        ],
        out_specs=pl.BlockSpec((n_block, HD), row),
        out_shape=jax.ShapeDtypeStruct((n, HD), jnp.float32),
    )(skip, a0, a1, d0, d1, erep, wa, wb, gb, gamma, beta, pa)
    return out


def kernel(feat, edge_index, Wq, bq, Wk, bk, Wv, bv, Ws, bs, Wg, bg, gamma,
           beta, prelu_a):
    n = feat.shape[0]
    e = edge_index.shape[1]
    npad = -(-n // (NS * zrows_unit)) * (NS * zrows_unit)
    isd = 1.0 / (D ** 0.5)
    wt = jnp.concatenate([Wq * isd, Wk, Wv, Ws], axis=0).T
    ball = jnp.concatenate([bq * isd, bk, bv, bs]).reshape(1, 4 * HD)
    feat_pad = jnp.pad(feat, ((0, npad - n), (0, 0)))
    q, k, v, skip = _qkvs(feat_pad, wt, ball, n_block=npad // 10)

    ept = -(-(e // NW) // EPC) * EPC
    nch = ept // EPC
    sd_flat = (edge_index[0].astype(jnp.int32)
               | (edge_index[1].astype(jnp.int32) << 16))
    sd_flat = jnp.pad(sd_flat, (0, NW * ept - e),
                      constant_values=int(n) << 16)
    sd = sd_flat.reshape(NW, nch, EPC)
    a0, a1, d0, d1 = _edge_sc(sd, q, k, v, npad, e)

    wg3 = Wg.reshape(3, HD)
    wa = (wg3[0] + wg3[2]).reshape(1, HD)
    wb = (wg3[1] - wg3[2]).reshape(1, HD)
    erep = (jnp.arange(HD)[None, :] // D == jnp.arange(16)[:, None]
            ).astype(jnp.float32)
    gb = bg.reshape(1, 1)
    pa = jnp.reshape(prelu_a, (1, 1))
    return _epilogue(skip, a0, a1, d0, d1, erep, wa, wb, gb,
                     gamma.reshape(1, HD), beta.reshape(1, HD), pa,
                     n_block=2000, n_out=n)


# speedup vs baseline: 1.0037x; 1.0037x over previous
import functools

import jax
import jax.numpy as jnp
from jax import lax
from jax.experimental import pallas as pl
from jax.experimental.pallas import tpu as pltpu
from jax.experimental.pallas import tpu_sc as plsc

H = 8
D = 16
HD = H * D

NC = 2
NS = 16
NW = NC * NS
EPC = 32
zrows_unit = 16


def _qkvs_body(x_ref, w_ref, b_ref, q_ref, k_ref, v_ref, s_ref):
    y = jnp.dot(x_ref[...], w_ref[...], preferred_element_type=jnp.float32)
    y = y + b_ref[...]
    q_ref[...] = y[:, 0 * HD:1 * HD]
    k_ref[...] = y[:, 1 * HD:2 * HD]
    v_ref[...] = y[:, 2 * HD:3 * HD]
    s_ref[...] = y[:, 3 * HD:4 * HD]


def _qkvs(feat, wt, ball, n_block):
    n = feat.shape[0]
    grid = (n // n_block,)
    spec_x = pl.BlockSpec((n_block, HD), lambda i: (i, 0))
    spec_w = pl.BlockSpec((HD, 4 * HD), lambda i: (0, 0))
    spec_b = pl.BlockSpec((1, 4 * HD), lambda i: (0, 0))
    spec_o = pl.BlockSpec((n_block, HD), lambda i: (i, 0))
    out = pl.pallas_call(
        _qkvs_body,
        grid=grid,
        in_specs=[spec_x, spec_w, spec_b],
        out_specs=[spec_o] * 4,
        out_shape=[jax.ShapeDtypeStruct((n, HD), jnp.float32)] * 4,
    )(feat, wt, ball)
    return out


def _edge_sc(sd, q, k, v, npad, e):
    epc = EPC
    nch = sd.shape[1]
    rows_per_tile = npad // NS
    zrows = zrows_unit
    nzcop = rows_per_tile // zrows
    mesh = plsc.VectorSubcoreMesh(core_axis_name="c", subcore_axis_name="s")

    @functools.partial(
        pl.kernel,
        mesh=mesh,
        compiler_params=pltpu.CompilerParams(
            needs_layout_passes=False, use_tc_tiling_on_sc=False),
        out_type=[
            jax.ShapeDtypeStruct((npad, HD), jnp.float32),
            jax.ShapeDtypeStruct((npad, HD), jnp.float32),
            jax.ShapeDtypeStruct((npad, 16), jnp.float32),
            jax.ShapeDtypeStruct((npad, 16), jnp.float32),
        ],
        scratch_types=[
            pltpu.VMEM((nch, epc), jnp.int32),
            pltpu.VMEM((epc,), jnp.int32),
            pltpu.VMEM((2, epc), jnp.int32),
            pltpu.VMEM((epc, HD), jnp.float32),
            pltpu.VMEM((epc, HD), jnp.float32),
            pltpu.VMEM((epc, HD), jnp.float32),
            pltpu.VMEM((epc, 16), jnp.float32),
            pltpu.VMEM((zrows_unit, HD), jnp.float32),
            pltpu.VMEM((zrows_unit, 16), jnp.float32),
            pltpu.VMEM_SHARED((npad, HD), jnp.float32),
            pltpu.VMEM_SHARED((npad, 16), jnp.float32),
            pltpu.SemaphoreType.DMA,
            pltpu.SemaphoreType.DMA,
        ],
    )
    def edge_kernel(sd_hbm, q_hbm, k_hbm, v_hbm,
                    agg0_out, agg1_out, den0_out, den1_out,
                    sd_v, sidx, didx2, qrows, krows, vrows, exv, zbuf, zbufd,
                    agg_s, den_s, sem_g, sem_s):
        c = lax.axis_index("c")
        s = lax.axis_index("s")
        w = c * NS + s
        rbase = s * rows_per_tile
        z16 = jnp.zeros((16,), jnp.float32)
        z8 = jnp.zeros((8,), jnp.float32)

        def zero_body(i, carry):
            for j in range(HD // 16):
                zbuf[i, pl.ds(j * 16, 16)] = z16
            return carry

        lax.fori_loop(0, zrows, zero_body, 0)
        for kk in range(nzcop):
            sl = pl.ds(rbase + kk * zrows, zrows)
            pltpu.sync_copy(zbuf, agg_s.at[sl])
            pltpu.sync_copy(zbuf.at[pl.ds(0, zrows), pl.ds(0, 16)],
                            den_s.at[sl])
        plsc.subcore_barrier()

        pltpu.sync_copy(sd_hbm.at[w], sd_v)

        lane = lax.iota(jnp.int32, 16)
        m0 = lane == jnp.zeros((16,), jnp.int32)

        def ez_body(i, carry):
            exv[i, :] = z16
            return carry

        lax.fori_loop(0, epc, ez_body, 0)

        def chunk_body(g, carry):
            p = lax.rem(g, 2)
            didx = didx2.at[p]
            for tt in range(epc // 16):
                sv = sd_v[g, pl.ds(tt * 16, 16)]
                sidx[pl.ds(tt * 16, 16)] = lax.bitwise_and(sv, 0xFFFF)
                didx[pl.ds(tt * 16, 16)] = lax.shift_right_logical(sv, 16)
            cq = pltpu.async_copy(q_hbm.at[sidx], qrows, sem_g)
            ck = pltpu.async_copy(k_hbm.at[didx], krows, sem_g)

            @pl.when(g > 0)
            def _():
                dprev = didx2.at[1 - p]
                pltpu.make_async_copy(exv, den_s.at[dprev], sem_s).wait()
                pltpu.make_async_copy(vrows, agg_s.at[dprev], sem_s).wait()

            cv = pltpu.async_copy(v_hbm.at[sidx], vrows, sem_g)
            cq.wait()
            ck.wait()
            cv.wait()
            def ee_body(ee, carry):
                eev = jnp.full((16,), ee, jnp.int32)
                for h in range(H):
                    qv = qrows[ee, pl.ds(h * D, D)]
                    kv = krows[ee, pl.ds(h * D, D)]
                    ev = jnp.exp(jnp.full((16,), jnp.sum(qv * kv),
                                          jnp.float32))
                    plsc.store_scatter(
                        exv,
                        [eev, jnp.full((16,), h, jnp.int32)],
                        ev, mask=m0)
                    vrows[ee, pl.ds(h * D, D)] = (
                        vrows[ee, pl.ds(h * D, D)] * ev)
                return carry

            lax.fori_loop(0, epc, ee_body, 0)
            pltpu.async_copy(exv, den_s.at[didx], sem_s, add=True)
            pltpu.async_copy(vrows, agg_s.at[didx], sem_s, add=True)
            return carry

        lax.fori_loop(0, nch, chunk_body, 0)
        dlast = didx2.at[(nch - 1) % 2]
        pltpu.make_async_copy(exv, den_s.at[dlast], sem_s).wait()
        pltpu.make_async_copy(vrows, agg_s.at[dlast], sem_s).wait()
        plsc.subcore_barrier()

        def dump_body(kk, carry):
            sl = pl.ds(rbase + kk * zrows, zrows)
            pltpu.sync_copy(agg_s.at[sl], zbuf)
            pltpu.sync_copy(den_s.at[sl], zbufd)

            @pl.when(c == 0)
            def _():
                pltpu.sync_copy(zbuf, agg0_out.at[sl])
                pltpu.sync_copy(zbufd, den0_out.at[sl])

            @pl.when(c == 1)
            def _():
                pltpu.sync_copy(zbuf, agg1_out.at[sl])
                pltpu.sync_copy(zbufd, den1_out.at[sl])

            return carry

        lax.fori_loop(0, nzcop, dump_body, 0)

    return edge_kernel(sd, q, k, v)


def _epi_body(skip_ref, a0_ref, a1_ref, d0_ref, d1_ref, erep_ref, wa_ref,
              wb_ref, gb_ref, gamma_ref, beta_ref, pa_ref, out_ref):
    skip = skip_ref[...]
    aggu = a0_ref[...] + a1_ref[...]
    den = d0_ref[...] + d1_ref[...]
    rec = jnp.where(den > 0.0, 1.0 / den, 0.0)
    recf = jnp.dot(rec, erep_ref[...], preferred_element_type=jnp.float32)
    agg = aggu * recf
    logit = (jnp.sum(skip * wa_ref[...], axis=-1, keepdims=True)
             + jnp.sum(agg * wb_ref[...], axis=-1, keepdims=True)
             + gb_ref[0, 0])
    gate = jax.nn.sigmoid(logit)
    rst = gate * skip + (1.0 - gate) * agg
    mu = jnp.mean(rst, axis=-1, keepdims=True)
    var = jnp.mean((rst - mu) * (rst - mu), axis=-1, keepdims=True)
    y = (rst - mu) * lax.rsqrt(var + 1e-5)
    y = y * gamma_ref[...] + beta_ref[...]
    out_ref[...] = jnp.where(y >= 0.0, y, pa_ref[0, 0] * y)


def _epilogue(skip, a0, a1, d0, d1, erep, wa, wb, gb, gamma, beta, pa, n_block,
              n_out):
    n = n_out
    grid = (n // n_block,)
    row = lambda i: (i, 0)
    full = lambda i: (0, 0)
    out = pl.pallas_call(
        _epi_body,
        grid=grid,
        in_specs=[
            pl.BlockSpec((n_block, HD), row),
            pl.BlockSpec((n_block, HD), row),
            pl.BlockSpec((n_block, HD), row),
            pl.BlockSpec((n_block, 16), row),
            pl.BlockSpec((n_block, 16), row),
            pl.BlockSpec((16, HD), full),
            pl.BlockSpec((1, HD), full),
            pl.BlockSpec((1, HD), full),
            pl.BlockSpec((1, 1), full),
            pl.BlockSpec((1, HD), full),
            pl.BlockSpec((1, HD), full),
            pl.BlockSpec((1, 1), full),
        ],
        out_specs=pl.BlockSpec((n_block, HD), row),
        out_shape=jax.ShapeDtypeStruct((n, HD), jnp.float32),
    )(skip, a0, a1, d0, d1, erep, wa, wb, gb, gamma, beta, pa)
    return out


def kernel(feat, edge_index, Wq, bq, Wk, bk, Wv, bv, Ws, bs, Wg, bg, gamma,
           beta, prelu_a):
    n = feat.shape[0]
    e = edge_index.shape[1]
    npad = -(-n // (NS * zrows_unit)) * (NS * zrows_unit)
    isd = 1.0 / (D ** 0.5)
    wt = jnp.concatenate([Wq * isd, Wk, Wv, Ws], axis=0).T
    ball = jnp.concatenate([bq * isd, bk, bv, bs]).reshape(1, 4 * HD)
    feat_pad = jnp.pad(feat, ((0, npad - n), (0, 0)))
    q, k, v, skip = _qkvs(feat_pad, wt, ball, n_block=npad // 10)

    ept = -(-(e // NW) // EPC) * EPC
    nch = ept // EPC
    sd_flat = (edge_index[0].astype(jnp.int32)
               | (edge_index[1].astype(jnp.int32) << 16))
    sd_flat = jnp.pad(sd_flat, (0, NW * ept - e),
                      constant_values=int(n) << 16)
    sd = sd_flat.reshape(NW, nch, EPC)
    a0, a1, d0, d1 = _edge_sc(sd, q, k, v, npad, e)

    wg3 = Wg.reshape(3, HD)
    wa = (wg3[0] + wg3[2]).reshape(1, HD)
    wb = (wg3[1] - wg3[2]).reshape(1, HD)
    erep = (jnp.arange(HD)[None, :] // D == jnp.arange(16)[:, None]
            ).astype(jnp.float32)
    gb = bg.reshape(1, 1)
    pa = jnp.reshape(prelu_a, (1, 1))
    return _epilogue(skip, a0, a1, d0, d1, erep, wa, wb, gb,
                     gamma.reshape(1, HD), beta.reshape(1, HD), pa,
                     n_block=2000, n_out=n)


# speedup vs baseline: 2.7181x; 2.7082x over previous
import functools

import jax
import jax.numpy as jnp
from jax import lax
from jax.experimental import pallas as pl
from jax.experimental.pallas import tpu as pltpu
from jax.experimental.pallas import tpu_sc as plsc

H = 8
D = 16
HD = H * D

NC = 2
NS = 16
NW = NC * NS
EPC = 32
zrows_unit = 16


def _qkvs_body(x_ref, w_ref, b_ref, q_ref, k_ref, v_ref, s_ref):
    y = jnp.dot(x_ref[...], w_ref[...], preferred_element_type=jnp.float32)
    y = y + b_ref[...]
    q_ref[...] = y[:, 0 * HD:1 * HD]
    k_ref[...] = y[:, 1 * HD:2 * HD]
    v_ref[...] = y[:, 2 * HD:3 * HD]
    s_ref[...] = y[:, 3 * HD:4 * HD]


def _qkvs(feat, wt, ball, n_block):
    n = feat.shape[0]
    grid = (n // n_block,)
    spec_x = pl.BlockSpec((n_block, HD), lambda i: (i, 0))
    spec_w = pl.BlockSpec((HD, 4 * HD), lambda i: (0, 0))
    spec_b = pl.BlockSpec((1, 4 * HD), lambda i: (0, 0))
    spec_o = pl.BlockSpec((n_block, HD), lambda i: (i, 0))
    out = pl.pallas_call(
        _qkvs_body,
        grid=grid,
        in_specs=[spec_x, spec_w, spec_b],
        out_specs=[spec_o] * 4,
        out_shape=[jax.ShapeDtypeStruct((n, HD), jnp.float32)] * 4,
    )(feat, wt, ball)
    return out


def _edge_sc(sd, q, k, v, npad, e):
    epc = EPC
    nch = sd.shape[1]
    rows_per_tile = npad // NS
    zrows = zrows_unit
    nzcop = rows_per_tile // zrows
    mesh = plsc.VectorSubcoreMesh(core_axis_name="c", subcore_axis_name="s")

    @functools.partial(
        pl.kernel,
        mesh=mesh,
        compiler_params=pltpu.CompilerParams(
            needs_layout_passes=False, use_tc_tiling_on_sc=False),
        out_type=[
            jax.ShapeDtypeStruct((npad, HD), jnp.float32),
            jax.ShapeDtypeStruct((npad, HD), jnp.float32),
            jax.ShapeDtypeStruct((npad, 16), jnp.float32),
            jax.ShapeDtypeStruct((npad, 16), jnp.float32),
        ],
        scratch_types=[
            pltpu.VMEM((nch, epc), jnp.int32),
            pltpu.VMEM((epc,), jnp.int32),
            pltpu.VMEM((2, epc), jnp.int32),
            pltpu.VMEM((epc, HD), jnp.float32),
            pltpu.VMEM((epc, HD), jnp.float32),
            pltpu.VMEM((epc, HD), jnp.float32),
            pltpu.VMEM((epc, 16), jnp.float32),
            pltpu.VMEM((zrows_unit, HD), jnp.float32),
            pltpu.VMEM((zrows_unit, 16), jnp.float32),
            pltpu.VMEM_SHARED((npad, HD), jnp.float32),
            pltpu.VMEM_SHARED((npad, 16), jnp.float32),
            pltpu.SemaphoreType.DMA,
            pltpu.SemaphoreType.DMA,
        ],
    )
    def edge_kernel(sd_hbm, q_hbm, k_hbm, v_hbm,
                    agg0_out, agg1_out, den0_out, den1_out,
                    sd_v, sidx, didx2, qrows, krows, vrows, exv, zbuf, zbufd,
                    agg_s, den_s, sem_g, sem_s):
        c = lax.axis_index("c")
        s = lax.axis_index("s")
        w = c * NS + s
        rbase = s * rows_per_tile
        z16 = jnp.zeros((16,), jnp.float32)
        z8 = jnp.zeros((8,), jnp.float32)

        def zero_body(i, carry):
            for j in range(HD // 16):
                zbuf[i, pl.ds(j * 16, 16)] = z16
            return carry

        lax.fori_loop(0, zrows, zero_body, 0)
        for kk in range(nzcop):
            sl = pl.ds(rbase + kk * zrows, zrows)
            pltpu.sync_copy(zbuf, agg_s.at[sl])
            pltpu.sync_copy(zbuf.at[pl.ds(0, zrows), pl.ds(0, 16)],
                            den_s.at[sl])
        plsc.subcore_barrier()

        pltpu.sync_copy(sd_hbm.at[w], sd_v)

        lane = lax.iota(jnp.int32, 16)
        px = [jnp.bitwise_xor(lane, 1 << b) for b in range(4)]
        oneh = [jnp.where(lane == jnp.full((16,), h, jnp.int32),
                          jnp.float32(1.0), jnp.float32(0.0))
                for h in range(H)]

        def chunk_body(g, carry):
            p = lax.rem(g, 2)
            didx = didx2.at[p]
            for tt in range(epc // 16):
                sv = sd_v[g, pl.ds(tt * 16, 16)]
                sidx[pl.ds(tt * 16, 16)] = lax.bitwise_and(sv, 0xFFFF)
                didx[pl.ds(tt * 16, 16)] = lax.shift_right_logical(sv, 16)
            cq = pltpu.async_copy(q_hbm.at[sidx], qrows, sem_g)
            ck = pltpu.async_copy(k_hbm.at[didx], krows, sem_g)

            @pl.when(g > 0)
            def _():
                dprev = didx2.at[1 - p]
                pltpu.make_async_copy(exv, den_s.at[dprev], sem_s).wait()
                pltpu.make_async_copy(vrows, agg_s.at[dprev], sem_s).wait()

            cv = pltpu.async_copy(v_hbm.at[sidx], vrows, sem_g)
            cq.wait()
            ck.wait()
            cv.wait()
            def ee_body(ee, carry):
                row = z16
                for h in range(H):
                    x = (qrows[ee, pl.ds(h * D, D)]
                         * krows[ee, pl.ds(h * D, D)])
                    for b in range(4):
                        x = x + jnp.take_along_axis(x, px[b], axis=0)
                    row = row + x * oneh[h]
                erow = jnp.exp(row)
                exv[ee, :] = erow
                for h in range(H):
                    ev = jnp.take_along_axis(
                        erow, jnp.full((16,), h, jnp.int32), axis=0)
                    vrows[ee, pl.ds(h * D, D)] = (
                        vrows[ee, pl.ds(h * D, D)] * ev)
                return carry

            lax.fori_loop(0, epc, ee_body, 0)
            pltpu.async_copy(exv, den_s.at[didx], sem_s, add=True)
            pltpu.async_copy(vrows, agg_s.at[didx], sem_s, add=True)
            return carry

        lax.fori_loop(0, nch, chunk_body, 0)
        dlast = didx2.at[(nch - 1) % 2]
        pltpu.make_async_copy(exv, den_s.at[dlast], sem_s).wait()
        pltpu.make_async_copy(vrows, agg_s.at[dlast], sem_s).wait()
        plsc.subcore_barrier()

        def dump_body(kk, carry):
            sl = pl.ds(rbase + kk * zrows, zrows)
            pltpu.sync_copy(agg_s.at[sl], zbuf)
            pltpu.sync_copy(den_s.at[sl], zbufd)

            @pl.when(c == 0)
            def _():
                pltpu.sync_copy(zbuf, agg0_out.at[sl])
                pltpu.sync_copy(zbufd, den0_out.at[sl])

            @pl.when(c == 1)
            def _():
                pltpu.sync_copy(zbuf, agg1_out.at[sl])
                pltpu.sync_copy(zbufd, den1_out.at[sl])

            return carry

        lax.fori_loop(0, nzcop, dump_body, 0)

    return edge_kernel(sd, q, k, v)


def _epi_body(skip_ref, a0_ref, a1_ref, d0_ref, d1_ref, erep_ref, wa_ref,
              wb_ref, gb_ref, gamma_ref, beta_ref, pa_ref, out_ref):
    skip = skip_ref[...]
    aggu = a0_ref[...] + a1_ref[...]
    den = d0_ref[...] + d1_ref[...]
    rec = jnp.where(den > 0.0, 1.0 / den, 0.0)
    recf = jnp.dot(rec, erep_ref[...], preferred_element_type=jnp.float32)
    agg = aggu * recf
    logit = (jnp.sum(skip * wa_ref[...], axis=-1, keepdims=True)
             + jnp.sum(agg * wb_ref[...], axis=-1, keepdims=True)
             + gb_ref[0, 0])
    gate = jax.nn.sigmoid(logit)
    rst = gate * skip + (1.0 - gate) * agg
    mu = jnp.mean(rst, axis=-1, keepdims=True)
    var = jnp.mean((rst - mu) * (rst - mu), axis=-1, keepdims=True)
    y = (rst - mu) * lax.rsqrt(var + 1e-5)
    y = y * gamma_ref[...] + beta_ref[...]
    out_ref[...] = jnp.where(y >= 0.0, y, pa_ref[0, 0] * y)


def _epilogue(skip, a0, a1, d0, d1, erep, wa, wb, gb, gamma, beta, pa, n_block,
              n_out):
    n = n_out
    grid = (n // n_block,)
    row = lambda i: (i, 0)
    full = lambda i: (0, 0)
    out = pl.pallas_call(
        _epi_body,
        grid=grid,
        in_specs=[
            pl.BlockSpec((n_block, HD), row),
            pl.BlockSpec((n_block, HD), row),
            pl.BlockSpec((n_block, HD), row),
            pl.BlockSpec((n_block, 16), row),
            pl.BlockSpec((n_block, 16), row),
            pl.BlockSpec((16, HD), full),
            pl.BlockSpec((1, HD), full),
            pl.BlockSpec((1, HD), full),
            pl.BlockSpec((1, 1), full),
            pl.BlockSpec((1, HD), full),
            pl.BlockSpec((1, HD), full),
            pl.BlockSpec((1, 1), full),
        ],
        out_specs=pl.BlockSpec((n_block, HD), row),
        out_shape=jax.ShapeDtypeStruct((n, HD), jnp.float32),
    )(skip, a0, a1, d0, d1, erep, wa, wb, gb, gamma, beta, pa)
    return out


def kernel(feat, edge_index, Wq, bq, Wk, bk, Wv, bv, Ws, bs, Wg, bg, gamma,
           beta, prelu_a):
    n = feat.shape[0]
    e = edge_index.shape[1]
    npad = -(-n // (NS * zrows_unit)) * (NS * zrows_unit)
    isd = 1.0 / (D ** 0.5)
    wt = jnp.concatenate([Wq * isd, Wk, Wv, Ws], axis=0).T
    ball = jnp.concatenate([bq * isd, bk, bv, bs]).reshape(1, 4 * HD)
    feat_pad = jnp.pad(feat, ((0, npad - n), (0, 0)))
    q, k, v, skip = _qkvs(feat_pad, wt, ball, n_block=npad // 10)

    ept = -(-(e // NW) // EPC) * EPC
    nch = ept // EPC
    sd_flat = (edge_index[0].astype(jnp.int32)
               | (edge_index[1].astype(jnp.int32) << 16))
    sd_flat = jnp.pad(sd_flat, (0, NW * ept - e),
                      constant_values=int(n) << 16)
    sd = sd_flat.reshape(NW, nch, EPC)
    a0, a1, d0, d1 = _edge_sc(sd, q, k, v, npad, e)

    wg3 = Wg.reshape(3, HD)
    wa = (wg3[0] + wg3[2]).reshape(1, HD)
    wb = (wg3[1] - wg3[2]).reshape(1, HD)
    erep = (jnp.arange(HD)[None, :] // D == jnp.arange(16)[:, None]
            ).astype(jnp.float32)
    gb = bg.reshape(1, 1)
    pa = jnp.reshape(prelu_a, (1, 1))
    return _epilogue(skip, a0, a1, d0, d1, erep, wa, wb, gb,
                     gamma.reshape(1, HD), beta.reshape(1, HD), pa,
                     n_block=2000, n_out=n)


# speedup vs baseline: 2.7693x; 1.0188x over previous
import functools

import jax
import jax.numpy as jnp
from jax import lax
from jax.experimental import pallas as pl
from jax.experimental.pallas import tpu as pltpu
from jax.experimental.pallas import tpu_sc as plsc

H = 8
D = 16
HD = H * D

NC = 2
NS = 16
NW = NC * NS
EPC = 32
zrows_unit = 16


def _qkvs_body(x_ref, w_ref, b_ref, q_ref, k_ref, v_ref, s_ref):
    y = jnp.dot(x_ref[...], w_ref[...], preferred_element_type=jnp.float32)
    y = y + b_ref[...]
    q_ref[...] = y[:, 0 * HD:1 * HD]
    k_ref[...] = y[:, 1 * HD:2 * HD]
    v_ref[...] = y[:, 2 * HD:3 * HD]
    s_ref[...] = y[:, 3 * HD:4 * HD]


def _qkvs(feat, wt, ball, n_block):
    n = feat.shape[0]
    grid = (n // n_block,)
    spec_x = pl.BlockSpec((n_block, HD), lambda i: (i, 0))
    spec_w = pl.BlockSpec((HD, 4 * HD), lambda i: (0, 0))
    spec_b = pl.BlockSpec((1, 4 * HD), lambda i: (0, 0))
    spec_o = pl.BlockSpec((n_block, HD), lambda i: (i, 0))
    out = pl.pallas_call(
        _qkvs_body,
        grid=grid,
        in_specs=[spec_x, spec_w, spec_b],
        out_specs=[spec_o] * 4,
        out_shape=[jax.ShapeDtypeStruct((n, HD), jnp.float32)] * 4,
    )(feat, wt, ball)
    return out


def _edge_sc(sd, q, k, v, npad, e):
    epc = EPC
    nch = sd.shape[1]
    rows_per_tile = npad // NS
    zrows = zrows_unit
    nzcop = rows_per_tile // zrows
    mesh = plsc.VectorSubcoreMesh(core_axis_name="c", subcore_axis_name="s")

    @functools.partial(
        pl.kernel,
        mesh=mesh,
        compiler_params=pltpu.CompilerParams(
            needs_layout_passes=False, use_tc_tiling_on_sc=False),
        out_type=[
            jax.ShapeDtypeStruct((npad, HD), jnp.float32),
            jax.ShapeDtypeStruct((npad, HD), jnp.float32),
            jax.ShapeDtypeStruct((npad, 16), jnp.float32),
            jax.ShapeDtypeStruct((npad, 16), jnp.float32),
        ],
        scratch_types=[
            pltpu.VMEM((nch, epc), jnp.int32),
            pltpu.VMEM((epc,), jnp.int32),
            pltpu.VMEM((2, epc), jnp.int32),
            pltpu.VMEM((epc, HD), jnp.float32),
            pltpu.VMEM((epc, HD), jnp.float32),
            pltpu.VMEM((epc, HD), jnp.float32),
            pltpu.VMEM((epc, 16), jnp.float32),
            pltpu.VMEM((zrows_unit, HD), jnp.float32),
            pltpu.VMEM((zrows_unit, 16), jnp.float32),
            pltpu.VMEM_SHARED((npad, HD), jnp.float32),
            pltpu.VMEM_SHARED((npad, 16), jnp.float32),
            pltpu.SemaphoreType.DMA,
            pltpu.SemaphoreType.DMA,
        ],
    )
    def edge_kernel(sd_hbm, q_hbm, k_hbm, v_hbm,
                    agg0_out, agg1_out, den0_out, den1_out,
                    sd_v, sidx, didx2, qrows, krows, vrows, exv, zbuf, zbufd,
                    agg_s, den_s, sem_g, sem_s):
        c = lax.axis_index("c")
        s = lax.axis_index("s")
        w = c * NS + s
        rbase = s * rows_per_tile
        z16 = jnp.zeros((16,), jnp.float32)
        z8 = jnp.zeros((8,), jnp.float32)

        def zero_body(i, carry):
            for j in range(HD // 16):
                zbuf[i, pl.ds(j * 16, 16)] = z16
            return carry

        lax.fori_loop(0, zrows, zero_body, 0)
        for kk in range(nzcop):
            sl = pl.ds(rbase + kk * zrows, zrows)
            pltpu.sync_copy(zbuf, agg_s.at[sl])
            pltpu.sync_copy(zbuf.at[pl.ds(0, zrows), pl.ds(0, 16)],
                            den_s.at[sl])
        plsc.subcore_barrier()

        pltpu.sync_copy(sd_hbm.at[w], sd_v)

        lane = lax.iota(jnp.int32, 16)
        px = [jnp.bitwise_xor(lane, 1 << b) for b in range(4)]
        oneh = [jnp.where(lane == jnp.full((16,), h, jnp.int32),
                          jnp.float32(1.0), jnp.float32(0.0))
                for h in range(H)]

        def chunk_body(g, carry):
            p = lax.rem(g, 2)
            didx = didx2.at[p]
            for tt in range(epc // 16):
                sv = sd_v[g, pl.ds(tt * 16, 16)]
                sidx[pl.ds(tt * 16, 16)] = lax.bitwise_and(sv, 0xFFFF)
                didx[pl.ds(tt * 16, 16)] = lax.shift_right_logical(sv, 16)
            cq = pltpu.async_copy(q_hbm.at[sidx], qrows, sem_g)
            ck = pltpu.async_copy(k_hbm.at[didx], krows, sem_g)

            @pl.when(g > 0)
            def _():
                dprev = didx2.at[1 - p]
                pltpu.make_async_copy(exv, den_s.at[dprev], sem_s).wait()
                pltpu.make_async_copy(vrows, agg_s.at[dprev], sem_s).wait()

            cv = pltpu.async_copy(v_hbm.at[sidx], vrows, sem_g)
            cq.wait()
            ck.wait()
            def dot_body(ee, carry):
                row = z16
                for h in range(H):
                    x = (qrows[ee, pl.ds(h * D, D)]
                         * krows[ee, pl.ds(h * D, D)])
                    for b in range(4):
                        x = x + jnp.take_along_axis(x, px[b], axis=0)
                    row = row + x * oneh[h]
                exv[ee, :] = jnp.exp(row)
                return carry

            lax.fori_loop(0, epc, dot_body, 0)
            cv.wait()

            def scale_body(ee, carry):
                erow = exv[ee, :]
                for h in range(H):
                    ev = jnp.take_along_axis(
                        erow, jnp.full((16,), h, jnp.int32), axis=0)
                    vrows[ee, pl.ds(h * D, D)] = (
                        vrows[ee, pl.ds(h * D, D)] * ev)
                return carry

            lax.fori_loop(0, epc, scale_body, 0)
            pltpu.async_copy(exv, den_s.at[didx], sem_s, add=True)
            pltpu.async_copy(vrows, agg_s.at[didx], sem_s, add=True)
            return carry

        lax.fori_loop(0, nch, chunk_body, 0)
        dlast = didx2.at[(nch - 1) % 2]
        pltpu.make_async_copy(exv, den_s.at[dlast], sem_s).wait()
        pltpu.make_async_copy(vrows, agg_s.at[dlast], sem_s).wait()
        plsc.subcore_barrier()

        def dump_body(kk, carry):
            sl = pl.ds(rbase + kk * zrows, zrows)
            pltpu.sync_copy(agg_s.at[sl], zbuf)
            pltpu.sync_copy(den_s.at[sl], zbufd)

            @pl.when(c == 0)
            def _():
                pltpu.sync_copy(zbuf, agg0_out.at[sl])
                pltpu.sync_copy(zbufd, den0_out.at[sl])

            @pl.when(c == 1)
            def _():
                pltpu.sync_copy(zbuf, agg1_out.at[sl])
                pltpu.sync_copy(zbufd, den1_out.at[sl])

            return carry

        lax.fori_loop(0, nzcop, dump_body, 0)

    return edge_kernel(sd, q, k, v)


def _epi_body(skip_ref, a0_ref, a1_ref, d0_ref, d1_ref, erep_ref, wa_ref,
              wb_ref, gb_ref, gamma_ref, beta_ref, pa_ref, out_ref):
    skip = skip_ref[...]
    aggu = a0_ref[...] + a1_ref[...]
    den = d0_ref[...] + d1_ref[...]
    rec = jnp.where(den > 0.0, 1.0 / den, 0.0)
    recf = jnp.dot(rec, erep_ref[...], preferred_element_type=jnp.float32)
    agg = aggu * recf
    logit = (jnp.sum(skip * wa_ref[...], axis=-1, keepdims=True)
             + jnp.sum(agg * wb_ref[...], axis=-1, keepdims=True)
             + gb_ref[0, 0])
    gate = jax.nn.sigmoid(logit)
    rst = gate * skip + (1.0 - gate) * agg
    mu = jnp.mean(rst, axis=-1, keepdims=True)
    var = jnp.mean((rst - mu) * (rst - mu), axis=-1, keepdims=True)
    y = (rst - mu) * lax.rsqrt(var + 1e-5)
    y = y * gamma_ref[...] + beta_ref[...]
    out_ref[...] = jnp.where(y >= 0.0, y, pa_ref[0, 0] * y)


def _epilogue(skip, a0, a1, d0, d1, erep, wa, wb, gb, gamma, beta, pa, n_block,
              n_out):
    n = n_out
    grid = (n // n_block,)
    row = lambda i: (i, 0)
    full = lambda i: (0, 0)
    out = pl.pallas_call(
        _epi_body,
        grid=grid,
        in_specs=[
            pl.BlockSpec((n_block, HD), row),
            pl.BlockSpec((n_block, HD), row),
            pl.BlockSpec((n_block, HD), row),
            pl.BlockSpec((n_block, 16), row),
            pl.BlockSpec((n_block, 16), row),
            pl.BlockSpec((16, HD), full),
            pl.BlockSpec((1, HD), full),
            pl.BlockSpec((1, HD), full),
            pl.BlockSpec((1, 1), full),
            pl.BlockSpec((1, HD), full),
            pl.BlockSpec((1, HD), full),
            pl.BlockSpec((1, 1), full),
        ],
        out_specs=pl.BlockSpec((n_block, HD), row),
        out_shape=jax.ShapeDtypeStruct((n, HD), jnp.float32),
    )(skip, a0, a1, d0, d1, erep, wa, wb, gb, gamma, beta, pa)
    return out


def kernel(feat, edge_index, Wq, bq, Wk, bk, Wv, bv, Ws, bs, Wg, bg, gamma,
           beta, prelu_a):
    n = feat.shape[0]
    e = edge_index.shape[1]
    npad = -(-n // (NS * zrows_unit)) * (NS * zrows_unit)
    isd = 1.0 / (D ** 0.5)
    wt = jnp.concatenate([Wq * isd, Wk, Wv, Ws], axis=0).T
    ball = jnp.concatenate([bq * isd, bk, bv, bs]).reshape(1, 4 * HD)
    feat_pad = jnp.pad(feat, ((0, npad - n), (0, 0)))
    q, k, v, skip = _qkvs(feat_pad, wt, ball, n_block=npad // 10)

    ept = -(-(e // NW) // EPC) * EPC
    nch = ept // EPC
    sd_flat = (edge_index[0].astype(jnp.int32)
               | (edge_index[1].astype(jnp.int32) << 16))
    sd_flat = jnp.pad(sd_flat, (0, NW * ept - e),
                      constant_values=int(n) << 16)
    sd = sd_flat.reshape(NW, nch, EPC)
    a0, a1, d0, d1 = _edge_sc(sd, q, k, v, npad, e)

    wg3 = Wg.reshape(3, HD)
    wa = (wg3[0] + wg3[2]).reshape(1, HD)
    wb = (wg3[1] - wg3[2]).reshape(1, HD)
    erep = (jnp.arange(HD)[None, :] // D == jnp.arange(16)[:, None]
            ).astype(jnp.float32)
    gb = bg.reshape(1, 1)
    pa = jnp.reshape(prelu_a, (1, 1))
    return _epilogue(skip, a0, a1, d0, d1, erep, wa, wb, gb,
                     gamma.reshape(1, HD), beta.reshape(1, HD), pa,
                     n_block=2000, n_out=n)


# speedup vs baseline: 2.7710x; 1.0006x over previous
import functools

import jax
import jax.numpy as jnp
from jax import lax
from jax.experimental import pallas as pl
from jax.experimental.pallas import tpu as pltpu
from jax.experimental.pallas import tpu_sc as plsc

H = 8
D = 16
HD = H * D

NC = 2
NS = 16
NW = NC * NS
EPC = 32
zrows_unit = 16


def _qkvs_body(x_ref, w_ref, b_ref, q_ref, k_ref, v_ref, s_ref):
    y = jnp.dot(x_ref[...], w_ref[...], preferred_element_type=jnp.float32)
    y = y + b_ref[...]
    q_ref[...] = y[:, 0 * HD:1 * HD]
    k_ref[...] = y[:, 1 * HD:2 * HD]
    v_ref[...] = y[:, 2 * HD:3 * HD]
    s_ref[...] = y[:, 3 * HD:4 * HD]


def _qkvs(feat, wt, ball, n_block):
    n = feat.shape[0]
    grid = (n // n_block,)
    spec_x = pl.BlockSpec((n_block, HD), lambda i: (i, 0))
    spec_w = pl.BlockSpec((HD, 4 * HD), lambda i: (0, 0))
    spec_b = pl.BlockSpec((1, 4 * HD), lambda i: (0, 0))
    spec_o = pl.BlockSpec((n_block, HD), lambda i: (i, 0))
    out = pl.pallas_call(
        _qkvs_body,
        grid=grid,
        in_specs=[spec_x, spec_w, spec_b],
        out_specs=[spec_o] * 4,
        out_shape=[jax.ShapeDtypeStruct((n, HD), jnp.float32)] * 4,
    )(feat, wt, ball)
    return out


def _edge_sc(sd, q, k, v, npad, e):
    epc = EPC
    nch = sd.shape[1]
    rows_per_tile = npad // NS
    zrows = zrows_unit
    nzcop = rows_per_tile // zrows
    mesh = plsc.VectorSubcoreMesh(core_axis_name="c", subcore_axis_name="s")

    @functools.partial(
        pl.kernel,
        mesh=mesh,
        compiler_params=pltpu.CompilerParams(
            needs_layout_passes=False, use_tc_tiling_on_sc=False),
        out_type=[
            jax.ShapeDtypeStruct((npad, HD), jnp.float32),
            jax.ShapeDtypeStruct((npad, HD), jnp.float32),
            jax.ShapeDtypeStruct((npad, 16), jnp.float32),
            jax.ShapeDtypeStruct((npad, 16), jnp.float32),
        ],
        scratch_types=[
            pltpu.VMEM((nch, epc), jnp.int32),
            pltpu.VMEM((epc,), jnp.int32),
            pltpu.VMEM((2, epc), jnp.int32),
            pltpu.VMEM((epc, HD), jnp.float32),
            pltpu.VMEM((epc, HD), jnp.float32),
            pltpu.VMEM((epc, HD), jnp.float32),
            pltpu.VMEM((epc, 16), jnp.float32),
            pltpu.VMEM((zrows_unit, HD), jnp.float32),
            pltpu.VMEM((zrows_unit, 16), jnp.float32),
            pltpu.VMEM_SHARED((npad, HD), jnp.float32),
            pltpu.VMEM_SHARED((npad, 16), jnp.float32),
            pltpu.SemaphoreType.DMA,
            pltpu.SemaphoreType.DMA,
        ],
    )
    def edge_kernel(sd_hbm, q_hbm, k_hbm, v_hbm,
                    agg0_out, agg1_out, den0_out, den1_out,
                    sd_v, sidx, didx2, qrows, krows, vrows, exv, zbuf, zbufd,
                    agg_s, den_s, sem_g, sem_s):
        c = lax.axis_index("c")
        s = lax.axis_index("s")
        w = c * NS + s
        rbase = s * rows_per_tile
        z16 = jnp.zeros((16,), jnp.float32)

        def zero_body(i, carry):
            for j in range(HD // 16):
                zbuf[i, pl.ds(j * 16, 16)] = z16
            return carry

        lax.fori_loop(0, zrows, zero_body, 0)
        for kk in range(nzcop):
            sl = pl.ds(rbase + kk * zrows, zrows)
            pltpu.sync_copy(zbuf, agg_s.at[sl])
            pltpu.sync_copy(zbuf.at[pl.ds(0, zrows), pl.ds(0, 16)],
                            den_s.at[sl])
        plsc.subcore_barrier()

        pltpu.sync_copy(sd_hbm.at[w], sd_v)

        lane = lax.iota(jnp.int32, 16)
        px = [jnp.bitwise_xor(lane, 1 << b) for b in range(4)]
        oneh = [jnp.where(lane == jnp.full((16,), h, jnp.int32),
                          jnp.float32(1.0), jnp.float32(0.0))
                for h in range(H)]

        def chunk_body(g, carry):
            p = lax.rem(g, 2)
            didx = didx2.at[p]
            for tt in range(epc // 16):
                sv = sd_v[g, pl.ds(tt * 16, 16)]
                sidx[pl.ds(tt * 16, 16)] = lax.bitwise_and(sv, 0xFFFF)
                didx[pl.ds(tt * 16, 16)] = lax.shift_right_logical(sv, 16)
            cq = pltpu.async_copy(q_hbm.at[sidx], qrows, sem_g)
            ck = pltpu.async_copy(k_hbm.at[didx], krows, sem_g)

            @pl.when(g > 0)
            def _():
                dprev = didx2.at[1 - p]
                pltpu.make_async_copy(exv, den_s.at[dprev], sem_s).wait()
                pltpu.make_async_copy(vrows, agg_s.at[dprev], sem_s).wait()

            cv = pltpu.async_copy(v_hbm.at[sidx], vrows, sem_g)
            cq.wait()
            ck.wait()
            def dot_body(ee, carry):
                row = z16
                for h in range(H):
                    x = (qrows[ee, pl.ds(h * D, D)]
                         * krows[ee, pl.ds(h * D, D)])
                    for b in range(4):
                        x = x + jnp.take_along_axis(x, px[b], axis=0)
                    row = row + x * oneh[h]
                exv[ee, :] = jnp.exp(row)
                return carry

            lax.fori_loop(0, epc, dot_body, 0)
            cv.wait()

            def scale_body(ee, carry):
                erow = exv[ee, :]
                for h in range(H):
                    ev = jnp.take_along_axis(
                        erow, jnp.full((16,), h, jnp.int32), axis=0)
                    vrows[ee, pl.ds(h * D, D)] = (
                        vrows[ee, pl.ds(h * D, D)] * ev)
                return carry

            lax.fori_loop(0, epc, scale_body, 0)
            pltpu.async_copy(exv, den_s.at[didx], sem_s, add=True)
            pltpu.async_copy(vrows, agg_s.at[didx], sem_s, add=True)
            return carry

        lax.fori_loop(0, nch, chunk_body, 0)
        dlast = didx2.at[(nch - 1) % 2]
        pltpu.make_async_copy(exv, den_s.at[dlast], sem_s).wait()
        pltpu.make_async_copy(vrows, agg_s.at[dlast], sem_s).wait()
        plsc.subcore_barrier()

        def dump_body(kk, carry):
            sl = pl.ds(rbase + kk * zrows, zrows)
            pltpu.sync_copy(agg_s.at[sl], zbuf)
            pltpu.sync_copy(den_s.at[sl], zbufd)

            @pl.when(c == 0)
            def _():
                pltpu.sync_copy(zbuf, agg0_out.at[sl])
                pltpu.sync_copy(zbufd, den0_out.at[sl])

            @pl.when(c == 1)
            def _():
                pltpu.sync_copy(zbuf, agg1_out.at[sl])
                pltpu.sync_copy(zbufd, den1_out.at[sl])

            return carry

        lax.fori_loop(0, nzcop, dump_body, 0)

    return edge_kernel(sd, q, k, v)


def _epi_body(skip_ref, a0_ref, a1_ref, d0_ref, d1_ref, erep_ref, wa_ref,
              wb_ref, gb_ref, gamma_ref, beta_ref, pa_ref, out_ref):
    skip = skip_ref[...]
    aggu = a0_ref[...] + a1_ref[...]
    den = d0_ref[...] + d1_ref[...]
    rec = jnp.where(den > 0.0, 1.0 / den, 0.0)
    recf = jnp.dot(rec, erep_ref[...], preferred_element_type=jnp.float32)
    agg = aggu * recf
    logit = (jnp.sum(skip * wa_ref[...], axis=-1, keepdims=True)
             + jnp.sum(agg * wb_ref[...], axis=-1, keepdims=True)
             + gb_ref[0, 0])
    gate = jax.nn.sigmoid(logit)
    rst = gate * skip + (1.0 - gate) * agg
    mu = jnp.mean(rst, axis=-1, keepdims=True)
    var = jnp.mean((rst - mu) * (rst - mu), axis=-1, keepdims=True)
    y = (rst - mu) * lax.rsqrt(var + 1e-5)
    y = y * gamma_ref[...] + beta_ref[...]
    out_ref[...] = jnp.where(y >= 0.0, y, pa_ref[0, 0] * y)


def _epilogue(skip, a0, a1, d0, d1, erep, wa, wb, gb, gamma, beta, pa, n_block,
              n_out):
    n = n_out
    grid = (n // n_block,)
    row = lambda i: (i, 0)
    full = lambda i: (0, 0)
    out = pl.pallas_call(
        _epi_body,
        grid=grid,
        in_specs=[
            pl.BlockSpec((n_block, HD), row),
            pl.BlockSpec((n_block, HD), row),
            pl.BlockSpec((n_block, HD), row),
            pl.BlockSpec((n_block, 16), row),
            pl.BlockSpec((n_block, 16), row),
            pl.BlockSpec((16, HD), full),
            pl.BlockSpec((1, HD), full),
            pl.BlockSpec((1, HD), full),
            pl.BlockSpec((1, 1), full),
            pl.BlockSpec((1, HD), full),
            pl.BlockSpec((1, HD), full),
            pl.BlockSpec((1, 1), full),
        ],
        out_specs=pl.BlockSpec((n_block, HD), row),
        out_shape=jax.ShapeDtypeStruct((n, HD), jnp.float32),
    )(skip, a0, a1, d0, d1, erep, wa, wb, gb, gamma, beta, pa)
    return out


def kernel(feat, edge_index, Wq, bq, Wk, bk, Wv, bv, Ws, bs, Wg, bg, gamma,
           beta, prelu_a):
    n = feat.shape[0]
    e = edge_index.shape[1]
    npad = -(-n // (NS * zrows_unit)) * (NS * zrows_unit)
    isd = 1.0 / (D ** 0.5)
    wt = jnp.concatenate([Wq * isd, Wk, Wv, Ws], axis=0).T
    ball = jnp.concatenate([bq * isd, bk, bv, bs]).reshape(1, 4 * HD)
    feat_pad = jnp.pad(feat, ((0, npad - n), (0, 0)))
    q, k, v, skip = _qkvs(feat_pad, wt, ball, n_block=npad // 10)

    ept = -(-(e // NW) // EPC) * EPC
    nch = ept // EPC
    sd_flat = (edge_index[0].astype(jnp.int32)
               | (edge_index[1].astype(jnp.int32) << 16))
    sd_flat = jnp.pad(sd_flat, (0, NW * ept - e),
                      constant_values=int(n) << 16)
    sd = sd_flat.reshape(NW, nch, EPC)
    a0, a1, d0, d1 = _edge_sc(sd, q, k, v, npad, e)

    wg3 = Wg.reshape(3, HD)
    wa = (wg3[0] + wg3[2]).reshape(1, HD)
    wb = (wg3[1] - wg3[2]).reshape(1, HD)
    erep = (jnp.arange(HD)[None, :] // D == jnp.arange(16)[:, None]
            ).astype(jnp.float32)
    gb = bg.reshape(1, 1)
    pa = jnp.reshape(prelu_a, (1, 1))
    return _epilogue(skip, a0, a1, d0, d1, erep, wa, wb, gb,
                     gamma.reshape(1, HD), beta.reshape(1, HD), pa,
                     n_block=2000, n_out=n)


# speedup vs baseline: 3.1328x; 1.1306x over previous
import functools

import jax
import jax.numpy as jnp
from jax import lax
from jax.experimental import pallas as pl
from jax.experimental.pallas import tpu as pltpu
from jax.experimental.pallas import tpu_sc as plsc

H = 8
D = 16
HD = H * D

NC = 2
NS = 16
NW = NC * NS
EPC = 32
zrows_unit = 16


def _qkvs_body(x_ref, w_ref, b_ref, q_ref, k_ref, v_ref, s_ref):
    y = jnp.dot(x_ref[...], w_ref[...], preferred_element_type=jnp.float32)
    y = y + b_ref[...]
    q_ref[...] = y[:, 0 * HD:1 * HD]
    k_ref[...] = y[:, 1 * HD:2 * HD]
    v_ref[...] = y[:, 2 * HD:3 * HD]
    s_ref[...] = y[:, 3 * HD:4 * HD]


def _qkvs(feat, wt, ball, n_block):
    n = feat.shape[0]
    grid = (n // n_block,)
    spec_x = pl.BlockSpec((n_block, HD), lambda i: (i, 0))
    spec_w = pl.BlockSpec((HD, 4 * HD), lambda i: (0, 0))
    spec_b = pl.BlockSpec((1, 4 * HD), lambda i: (0, 0))
    spec_o = pl.BlockSpec((n_block, HD), lambda i: (i, 0))
    out = pl.pallas_call(
        _qkvs_body,
        grid=grid,
        in_specs=[spec_x, spec_w, spec_b],
        out_specs=[spec_o] * 4,
        out_shape=[jax.ShapeDtypeStruct((n, HD), jnp.float32)] * 4,
    )(feat, wt, ball)
    return out


def _edge_sc(sd, q, k, v, npad, e):
    epc = EPC
    nch = sd.shape[1]
    rows_per_tile = npad // NS
    zrows = zrows_unit
    nzcop = rows_per_tile // zrows
    mesh = plsc.VectorSubcoreMesh(core_axis_name="c", subcore_axis_name="s")

    @functools.partial(
        pl.kernel,
        mesh=mesh,
        compiler_params=pltpu.CompilerParams(
            needs_layout_passes=False, use_tc_tiling_on_sc=False),
        out_type=[
            jax.ShapeDtypeStruct((npad, HD), jnp.float32),
            jax.ShapeDtypeStruct((npad, HD), jnp.float32),
            jax.ShapeDtypeStruct((npad, 16), jnp.float32),
            jax.ShapeDtypeStruct((npad, 16), jnp.float32),
        ],
        scratch_types=[
            pltpu.VMEM((nch, epc), jnp.int32),
            pltpu.VMEM((2, epc), jnp.int32),
            pltpu.VMEM((2, epc), jnp.int32),
            pltpu.VMEM((epc, HD), jnp.float32),
            pltpu.VMEM((epc, HD), jnp.float32),
            pltpu.VMEM((epc, HD), jnp.float32),
            pltpu.VMEM((epc, 16), jnp.float32),
            pltpu.VMEM((zrows_unit, HD), jnp.float32),
            pltpu.VMEM((zrows_unit, 16), jnp.float32),
            pltpu.VMEM_SHARED((npad, HD), jnp.float32),
            pltpu.VMEM_SHARED((npad, 16), jnp.float32),
            pltpu.SemaphoreType.DMA,
            pltpu.SemaphoreType.DMA,
            pltpu.SemaphoreType.DMA,
        ],
    )
    def edge_kernel(sd_hbm, q_hbm, k_hbm, v_hbm,
                    agg0_out, agg1_out, den0_out, den1_out,
                    sd_v, sidx2, didx2, qrows, krows, vrows, exv, zbuf, zbufd,
                    agg_s, den_s, sem_g, sem_v, sem_s):
        c = lax.axis_index("c")
        s = lax.axis_index("s")
        w = c * NS + s
        rbase = s * rows_per_tile
        z16 = jnp.zeros((16,), jnp.float32)

        def zero_body(i, carry):
            for j in range(HD // 16):
                zbuf[i, pl.ds(j * 16, 16)] = z16
            return carry

        lax.fori_loop(0, zrows, zero_body, 0)
        for kk in range(nzcop):
            sl = pl.ds(rbase + kk * zrows, zrows)
            pltpu.sync_copy(zbuf, agg_s.at[sl])
            pltpu.sync_copy(zbuf.at[pl.ds(0, zrows), pl.ds(0, 16)],
                            den_s.at[sl])
        plsc.subcore_barrier()

        pltpu.sync_copy(sd_hbm.at[w], sd_v)

        lane = lax.iota(jnp.int32, 16)
        px = [jnp.bitwise_xor(lane, 1 << b) for b in range(4)]
        oneh = [jnp.where(lane == jnp.full((16,), h, jnp.int32),
                          jnp.float32(1.0), jnp.float32(0.0))
                for h in range(H)]

        def unpack(g, pp):
            sref = sidx2.at[pp]
            dref = didx2.at[pp]
            for tt in range(epc // 16):
                sv = sd_v[g, pl.ds(tt * 16, 16)]
                sref[pl.ds(tt * 16, 16)] = lax.bitwise_and(sv, 0xFFFF)
                dref[pl.ds(tt * 16, 16)] = lax.shift_right_logical(sv, 16)

        unpack(0, 0)
        pltpu.async_copy(q_hbm.at[sidx2.at[0]], qrows, sem_g)
        pltpu.async_copy(k_hbm.at[didx2.at[0]], krows, sem_g)

        def chunk_body(g, carry):
            p = lax.rem(g, 2)
            sidx = sidx2.at[p]
            didx = didx2.at[p]

            @pl.when(g > 0)
            def _():
                dprev = didx2.at[1 - p]
                pltpu.make_async_copy(exv, den_s.at[dprev], sem_s).wait()
                pltpu.make_async_copy(vrows, agg_s.at[dprev], sem_s).wait()

            cv = pltpu.async_copy(v_hbm.at[sidx], vrows, sem_v)
            pltpu.make_async_copy(q_hbm.at[sidx], qrows, sem_g).wait()
            pltpu.make_async_copy(k_hbm.at[didx], krows, sem_g).wait()
            def dot_body(ee, carry):
                row = z16
                for h in range(H):
                    x = (qrows[ee, pl.ds(h * D, D)]
                         * krows[ee, pl.ds(h * D, D)])
                    for b in range(4):
                        x = x + jnp.take_along_axis(x, px[b], axis=0)
                    row = row + x * oneh[h]
                exv[ee, :] = jnp.exp(row)
                return carry

            lax.fori_loop(0, epc, dot_body, 0)

            @pl.when(g < nch - 1)
            def _():
                unpack(g + 1, 1 - p)
                pltpu.async_copy(q_hbm.at[sidx2.at[1 - p]], qrows, sem_g)
                pltpu.async_copy(k_hbm.at[didx2.at[1 - p]], krows, sem_g)

            cv.wait()

            def scale_body(ee, carry):
                erow = exv[ee, :]
                for h in range(H):
                    ev = jnp.take_along_axis(
                        erow, jnp.full((16,), h, jnp.int32), axis=0)
                    vrows[ee, pl.ds(h * D, D)] = (
                        vrows[ee, pl.ds(h * D, D)] * ev)
                return carry

            lax.fori_loop(0, epc, scale_body, 0)
            pltpu.async_copy(exv, den_s.at[didx], sem_s, add=True)
            pltpu.async_copy(vrows, agg_s.at[didx], sem_s, add=True)
            return carry

        lax.fori_loop(0, nch, chunk_body, 0)
        dlast = didx2.at[(nch - 1) % 2]
        pltpu.make_async_copy(exv, den_s.at[dlast], sem_s).wait()
        pltpu.make_async_copy(vrows, agg_s.at[dlast], sem_s).wait()
        plsc.subcore_barrier()

        def dump_body(kk, carry):
            sl = pl.ds(rbase + kk * zrows, zrows)
            pltpu.sync_copy(agg_s.at[sl], zbuf)
            pltpu.sync_copy(den_s.at[sl], zbufd)

            @pl.when(c == 0)
            def _():
                pltpu.sync_copy(zbuf, agg0_out.at[sl])
                pltpu.sync_copy(zbufd, den0_out.at[sl])

            @pl.when(c == 1)
            def _():
                pltpu.sync_copy(zbuf, agg1_out.at[sl])
                pltpu.sync_copy(zbufd, den1_out.at[sl])

            return carry

        lax.fori_loop(0, nzcop, dump_body, 0)

    return edge_kernel(sd, q, k, v)


def _epi_body(skip_ref, a0_ref, a1_ref, d0_ref, d1_ref, erep_ref, wa_ref,
              wb_ref, gb_ref, gamma_ref, beta_ref, pa_ref, out_ref):
    skip = skip_ref[...]
    aggu = a0_ref[...] + a1_ref[...]
    den = d0_ref[...] + d1_ref[...]
    rec = jnp.where(den > 0.0, 1.0 / den, 0.0)
    recf = jnp.dot(rec, erep_ref[...], preferred_element_type=jnp.float32)
    agg = aggu * recf
    logit = (jnp.sum(skip * wa_ref[...], axis=-1, keepdims=True)
             + jnp.sum(agg * wb_ref[...], axis=-1, keepdims=True)
             + gb_ref[0, 0])
    gate = jax.nn.sigmoid(logit)
    rst = gate * skip + (1.0 - gate) * agg
    mu = jnp.mean(rst, axis=-1, keepdims=True)
    var = jnp.mean((rst - mu) * (rst - mu), axis=-1, keepdims=True)
    y = (rst - mu) * lax.rsqrt(var + 1e-5)
    y = y * gamma_ref[...] + beta_ref[...]
    out_ref[...] = jnp.where(y >= 0.0, y, pa_ref[0, 0] * y)


def _epilogue(skip, a0, a1, d0, d1, erep, wa, wb, gb, gamma, beta, pa, n_block,
              n_out):
    n = n_out
    grid = (n // n_block,)
    row = lambda i: (i, 0)
    full = lambda i: (0, 0)
    out = pl.pallas_call(
        _epi_body,
        grid=grid,
        in_specs=[
            pl.BlockSpec((n_block, HD), row),
            pl.BlockSpec((n_block, HD), row),
            pl.BlockSpec((n_block, HD), row),
            pl.BlockSpec((n_block, 16), row),
            pl.BlockSpec((n_block, 16), row),
            pl.BlockSpec((16, HD), full),
            pl.BlockSpec((1, HD), full),
            pl.BlockSpec((1, HD), full),
            pl.BlockSpec((1, 1), full),
            pl.BlockSpec((1, HD), full),
            pl.BlockSpec((1, HD), full),
            pl.BlockSpec((1, 1), full),
        ],
        out_specs=pl.BlockSpec((n_block, HD), row),
        out_shape=jax.ShapeDtypeStruct((n, HD), jnp.float32),
    )(skip, a0, a1, d0, d1, erep, wa, wb, gb, gamma, beta, pa)
    return out


def kernel(feat, edge_index, Wq, bq, Wk, bk, Wv, bv, Ws, bs, Wg, bg, gamma,
           beta, prelu_a):
    n = feat.shape[0]
    e = edge_index.shape[1]
    npad = -(-n // (NS * zrows_unit)) * (NS * zrows_unit)
    isd = 1.0 / (D ** 0.5)
    wt = jnp.concatenate([Wq * isd, Wk, Wv, Ws], axis=0).T
    ball = jnp.concatenate([bq * isd, bk, bv, bs]).reshape(1, 4 * HD)
    feat_pad = jnp.pad(feat, ((0, npad - n), (0, 0)))
    q, k, v, skip = _qkvs(feat_pad, wt, ball, n_block=npad // 10)

    ept = -(-(e // NW) // EPC) * EPC
    nch = ept // EPC
    sd_flat = (edge_index[0].astype(jnp.int32)
               | (edge_index[1].astype(jnp.int32) << 16))
    sd_flat = jnp.pad(sd_flat, (0, NW * ept - e),
                      constant_values=int(n) << 16)
    sd = sd_flat.reshape(NW, nch, EPC)
    a0, a1, d0, d1 = _edge_sc(sd, q, k, v, npad, e)

    wg3 = Wg.reshape(3, HD)
    wa = (wg3[0] + wg3[2]).reshape(1, HD)
    wb = (wg3[1] - wg3[2]).reshape(1, HD)
    erep = (jnp.arange(HD)[None, :] // D == jnp.arange(16)[:, None]
            ).astype(jnp.float32)
    gb = bg.reshape(1, 1)
    pa = jnp.reshape(prelu_a, (1, 1))
    return _epilogue(skip, a0, a1, d0, d1, erep, wa, wb, gb,
                     gamma.reshape(1, HD), beta.reshape(1, HD), pa,
                     n_block=2000, n_out=n)
